# Initial kernel scaffold; baseline (speedup 1.0000x reference)
import jax
import jax.numpy as jnp
from jax.experimental import pallas as pl


def kernel(x, edge_index, batch, params):
    raise NotImplementedError("write your pallas kernel here")


# speedup vs baseline: 4.9473x; 4.9473x over previous
import functools

import jax
import jax.numpy as jnp
from jax import lax
from jax.experimental import pallas as pl
from jax.experimental.pallas import tpu as pltpu
from jax.experimental.pallas import tpu_sc as plsc

N = 10000
D = 128
H = 128
G = 256
K = 6
E = 320000

NC = 2
NS = 16
NW = NC * NS
CH = 128
CHUNKS = -(-E // (NW * CH))
E_PAD = NW * CH * CHUNKS
N_DUMMY = N
N_PAD = ((N + 1 + 127) // 128) * 128
ZROWS = N_PAD // NS


def _sc_aggregate(h, src_p, dst_p, zeros_acc):
    mesh = plsc.VectorSubcoreMesh(core_axis_name="c", subcore_axis_name="s")

    @functools.partial(
        pl.kernel,
        out_type=jax.ShapeDtypeStruct((NC, N_PAD, H), jnp.float32),
        mesh=mesh,
        scratch_types=[
            pltpu.VMEM((CHUNKS, CH), jnp.int32),
            pltpu.VMEM((CHUNKS, CH), jnp.int32),
            pltpu.VMEM((CH, H), jnp.float32),
            pltpu.VMEM_SHARED((N_PAD, H), jnp.float32),
            pltpu.SemaphoreType.DMA,
        ],
    )
    def agg(h_hbm, src_hbm, dst_hbm, z_hbm, out_hbm, sidx_v, didx_v, rows_v,
            acc_sh, sem):
        cid = lax.axis_index("c")
        sid = lax.axis_index("s")
        wid = cid * NS + sid

        pltpu.sync_copy(z_hbm.at[pl.ds(sid * ZROWS, ZROWS)],
                        acc_sh.at[pl.ds(sid * ZROWS, ZROWS)])
        pltpu.sync_copy(src_hbm.at[wid], sidx_v)
        pltpu.sync_copy(dst_hbm.at[wid], didx_v)
        plsc.subcore_barrier()

        def body(j, carry):
            pltpu.async_copy(h_hbm.at[sidx_v.at[j]], rows_v, sem).wait()
            pltpu.sync_copy(rows_v, acc_sh.at[didx_v.at[j]], add=True)
            return carry

        lax.fori_loop(0, CHUNKS, body, 0, unroll=False)
        plsc.subcore_barrier()

        pltpu.sync_copy(acc_sh.at[pl.ds(sid * ZROWS, ZROWS)],
                        out_hbm.at[cid, pl.ds(sid * ZROWS, ZROWS)])

    return agg(h, src_p, dst_p, zeros_acc)


def _tc_layer(acc2, h, w_nbr, w_self, b):
    def body(acc_ref, h_ref, wn_ref, ws_ref, b_ref, o_ref):
        agg = acc_ref[0] + acc_ref[1]
        o = jnp.maximum(
            jnp.dot(agg, wn_ref[...], preferred_element_type=jnp.float32)
            + jnp.dot(h_ref[...], ws_ref[...], preferred_element_type=jnp.float32)
            + b_ref[...], 0.0)
        row = lax.broadcasted_iota(jnp.int32, (N_PAD, H), 0)
        o_ref[...] = jnp.where(row < N, o, 0.0)

    return pl.pallas_call(
        body,
        out_shape=jax.ShapeDtypeStruct((N_PAD, H), jnp.float32),
    )(acc2, h, w_nbr, w_self, b)


def _tc_pool(h, att_w, att_b, batch2d, heads_w8, heads_b8):
    def body(h_ref, aw_ref, ab_ref, batch_ref, hw_ref, hb_ref, o_ref):
        hv = h_ref[...]
        s = jnp.dot(hv, aw_ref[...],
                    preferred_element_type=jnp.float32)[:, 0] + ab_ref[0, 0]
        batch = batch_ref[0]
        valid = batch < G
        gids = lax.broadcasted_iota(jnp.int32, (G, N_PAD), 0)
        onehot = batch[None, :] == gids
        neg = jnp.float32(-jnp.inf)
        smax = jnp.max(jnp.where(onehot, s[None, :], neg), axis=1)
        smax_n = jnp.sum(jnp.where(onehot, smax[:, None], 0.0), axis=0)
        ex = jnp.where(valid, jnp.exp(s - smax_n), 0.0)
        denom = jnp.sum(jnp.where(onehot, ex[None, :], 0.0), axis=1)
        denom_n = jnp.sum(jnp.where(onehot, denom[:, None], 0.0), axis=0)
        alpha = jnp.where(valid, ex / jnp.maximum(denom_n, 1e-12), 0.0)
        m = onehot.astype(jnp.float32)
        pooled = jnp.dot(m, alpha[:, None] * hv,
                         preferred_element_type=jnp.float32)
        o_ref[...] = (jnp.dot(hw_ref[...], pooled.T,
                              preferred_element_type=jnp.float32)
                      + hb_ref[...])

    return pl.pallas_call(
        body,
        out_shape=jax.ShapeDtypeStruct((8, G), jnp.float32),
    )(h, att_w, att_b, batch2d, heads_w8, heads_b8)


def kernel(x, edge_index, batch, params):
    src, dst = edge_index[0], edge_index[1]
    pad = E_PAD - E
    src_p = jnp.concatenate(
        [src, jnp.zeros((pad,), jnp.int32)]).reshape(NW, CHUNKS, CH)
    dst_p = jnp.concatenate(
        [dst, jnp.full((pad,), N_DUMMY, jnp.int32)]).reshape(NW, CHUNKS, CH)
    zeros_acc = jnp.zeros((N_PAD, H), jnp.float32)

    h = jnp.zeros((N_PAD, H), jnp.float32).at[:N].set(x)
    for layer in params["layers"]:
        acc2 = _sc_aggregate(h, src_p, dst_p, zeros_acc)
        h = _tc_layer(acc2, h, layer["W_nbr"], layer["W_self"],
                      layer["b"].reshape(1, H))

    att_w = params["att_w"].reshape(H, 1)
    att_b = params["att_b"].reshape(1, 1)
    batch2d = jnp.full((1, N_PAD), G, jnp.int32).at[0, :N].set(batch)
    heads_w8 = jnp.zeros((8, H), jnp.float32).at[:K].set(params["heads_W"])
    heads_b8 = jnp.zeros((8, 1), jnp.float32).at[:K, 0].set(params["heads_b"])
    out8 = _tc_pool(h, att_w, att_b, batch2d, heads_w8, heads_b8)
    return out8[:K]
